# baseline (speedup 1.0000x reference)
import functools

import jax
import jax.numpy as jnp
from jax import lax
from jax.experimental import pallas as pl
from jax.experimental.pallas import tpu as pltpu
from jax.experimental.pallas import tpu_sc as plsc

NC = 2
NS = 16
NW = NC * NS
LANES = 16

CHUNK = 128


def _sc_mesh():
    return plsc.VectorSubcoreMesh(
        core_axis_name="c", subcore_axis_name="s", num_cores=NC, num_subcores=NS
    )


def _make_deg_kernel(n_pad, cpw):
    nps = n_pad // NS

    def body(srcs, dsts, deg_out, sidx, didx, ones_v, zbuf, deg_s, deg_d,
             sem0):
        c = lax.axis_index("c")
        s = lax.axis_index("s")
        wid = c * NS + s

        z16 = jnp.zeros((LANES,), jnp.float32)
        o16 = jnp.ones((LANES,), jnp.float32)

        def zb(i, _):
            zbuf[pl.ds(i * LANES, LANES)] = z16
            return 0

        lax.fori_loop(0, nps // LANES, zb, 0)

        def ob(i, _):
            ones_v[pl.ds(i * LANES, LANES)] = o16
            return 0

        lax.fori_loop(0, CHUNK // LANES, ob, 0)

        pltpu.sync_copy(zbuf, deg_s.at[pl.ds(s * nps, nps)])
        pltpu.sync_copy(zbuf, deg_d.at[pl.ds(s * nps, nps)])
        pltpu.sync_copy(srcs.at[wid], sidx)
        pltpu.sync_copy(dsts.at[wid], didx)
        plsc.subcore_barrier()

        grp = 8

        def step(q, _):
            for k in range(grp):
                g = q * grp + k
                pltpu.async_copy(ones_v, deg_s.at[sidx.at[g]], sem0, add=True)
                pltpu.async_copy(ones_v, deg_d.at[didx.at[g]], sem0, add=True)
            for k in range(grp):
                g = q * grp + k
                pltpu.make_async_copy(ones_v, deg_s.at[sidx.at[g]],
                                      sem0).wait()
                pltpu.make_async_copy(ones_v, deg_d.at[didx.at[g]],
                                      sem0).wait()
            return 0

        assert cpw % grp == 0
        lax.fori_loop(0, cpw // grp, step, 0)
        plsc.subcore_barrier()

        pltpu.sync_copy(deg_s.at[pl.ds(s * nps, nps)],
                        deg_out.at[2 * c, pl.ds(s * nps, nps)])
        pltpu.sync_copy(deg_d.at[pl.ds(s * nps, nps)],
                        deg_out.at[2 * c + 1, pl.ds(s * nps, nps)])

    return pl.kernel(
        body,
        out_type=jax.ShapeDtypeStruct((2 * NC, n_pad), jnp.float32),
        mesh=_sc_mesh(),
        scratch_types=[
            pltpu.VMEM((cpw, CHUNK), jnp.int32),
            pltpu.VMEM((cpw, CHUNK), jnp.int32),
            pltpu.VMEM((CHUNK,), jnp.float32),
            pltpu.VMEM((nps,), jnp.float32),
            pltpu.VMEM_SHARED((n_pad,), jnp.float32),
            pltpu.VMEM_SHARED((n_pad,), jnp.float32),
            pltpu.SemaphoreType.DMA,
        ],
    )


def _make_spmm_kernel(n_pad, d, cpw):
    nps = n_pad // NS
    passes = 2
    assert cpw % passes == 0 and (cpw // passes) % 2 == 0
    hcw = cpw // passes
    assert nps % CHUNK == 0

    def body(h_hbm, srcs, dsts, out_hbm, sidx, didx, buf0, buf1,
             agg, sem0, sem1, sems0, sems1):
        c = lax.axis_index("c")
        s = lax.axis_index("s")
        wid = c * NS + s

        z16 = jnp.zeros((LANES,), jnp.float32)

        def zb(i, _):
            r = i // (d // LANES)
            k = i % (d // LANES)
            buf0[r, pl.ds(k * LANES, LANES)] = z16
            return 0

        lax.fori_loop(0, CHUNK * (d // LANES), zb, 0)

        def zagg(k, _):
            pltpu.sync_copy(buf0, agg.at[pl.ds(s * nps + k * CHUNK, CHUNK)])
            return 0

        lax.fori_loop(0, nps // CHUNK, zagg, 0)
        plsc.subcore_barrier()

        for p in range(passes):
            pltpu.sync_copy(srcs.at[wid, pl.ds(p * hcw, hcw)], sidx)
            pltpu.sync_copy(dsts.at[wid, pl.ds(p * hcw, hcw)], didx)

            pltpu.async_copy(h_hbm.at[sidx.at[0]], buf0, sem0)
            pltpu.async_copy(h_hbm.at[sidx.at[1]], buf1, sem1)

            def step(g2, _):
                g = g2 * 2
                pltpu.make_async_copy(h_hbm.at[sidx.at[g]], buf0, sem0).wait()
                pltpu.async_copy(buf0, agg.at[didx.at[g]], sems0, add=True)
                pltpu.make_async_copy(h_hbm.at[sidx.at[g + 1]], buf1,
                                      sem1).wait()
                pltpu.async_copy(buf1, agg.at[didx.at[g + 1]], sems1, add=True)

                pltpu.make_async_copy(buf0, agg.at[didx.at[g]], sems0).wait()

                @pl.when(g + 2 < hcw)
                def _():
                    pltpu.async_copy(h_hbm.at[sidx.at[g + 2]], buf0, sem0)

                pltpu.make_async_copy(buf1, agg.at[didx.at[g + 1]],
                                      sems1).wait()

                @pl.when(g + 3 < hcw)
                def _():
                    pltpu.async_copy(h_hbm.at[sidx.at[g + 3]], buf1, sem1)

                return 0

            lax.fori_loop(0, hcw // 2, step, 0)

        plsc.subcore_barrier()
        pltpu.sync_copy(agg.at[pl.ds(s * nps, nps)],
                        out_hbm.at[c, pl.ds(s * nps, nps)])

    return pl.kernel(
        body,
        out_type=jax.ShapeDtypeStruct((NC, n_pad, d), jnp.float32),
        mesh=_sc_mesh(),
        scratch_types=[
            pltpu.VMEM((hcw, CHUNK), jnp.int32),
            pltpu.VMEM((hcw, CHUNK), jnp.int32),
            pltpu.VMEM((CHUNK, d), jnp.float32),
            pltpu.VMEM((CHUNK, d), jnp.float32),
            pltpu.VMEM_SHARED((n_pad, d), jnp.float32),
            pltpu.SemaphoreType.DMA,
            pltpu.SemaphoreType.DMA,
            pltpu.SemaphoreType.DMA,
            pltpu.SemaphoreType.DMA,
        ],
    )


def _norm_from(deg):
    return jnp.where(deg > 0, lax.rsqrt(jnp.maximum(deg, 1.0)), 0.0)


def _pre_body(x_ref, degt_ref, h_ref):
    deg_src = degt_ref[:, 0:1] + degt_ref[:, 2:3]
    h_ref[...] = x_ref[...] * _norm_from(deg_src)


def _post_body(parts_ref, degt_ref, w_ref, b_ref, out_ref, *, mid_layer, blk,
               n_real):
    agg = parts_ref[0] + parts_ref[1]
    deg_dst = degt_ref[:, 1:2] + degt_ref[:, 3:4]
    z = jnp.dot(agg * _norm_from(deg_dst), w_ref[...],
                preferred_element_type=jnp.float32) + b_ref[...]
    if mid_layer:
        z = jnp.maximum(z, 0.0)
        deg_src = degt_ref[:, 0:1] + degt_ref[:, 2:3]
        z = z * _norm_from(deg_src)
        rows = pl.program_id(0) * blk + lax.broadcasted_iota(
            jnp.int32, (blk, 1), 0)
        z = jnp.where(rows < n_real, z, 0.0)
    out_ref[...] = z


def _pre_call(x_p, degt, n_pad, d, blk=2048):
    grid = (n_pad // blk,)
    return pl.pallas_call(
        _pre_body,
        grid=grid,
        in_specs=[
            pl.BlockSpec((blk, d), lambda i: (i, 0)),
            pl.BlockSpec((blk, 2 * NC), lambda i: (i, 0)),
        ],
        out_specs=pl.BlockSpec((blk, d), lambda i: (i, 0)),
        out_shape=jax.ShapeDtypeStruct((n_pad, d), jnp.float32),
    )(x_p, degt)


def _post_call(parts, degt, w, b, *, mid_layer, n_real, n_pad, d, out_rows,
               blk):
    grid = (out_rows // blk,)
    body = functools.partial(_post_body, mid_layer=mid_layer, blk=blk,
                             n_real=n_real)
    return pl.pallas_call(
        body,
        grid=grid,
        in_specs=[
            pl.BlockSpec((NC, blk, d), lambda i: (0, i, 0)),
            pl.BlockSpec((blk, 2 * NC), lambda i: (i, 0)),
            pl.BlockSpec((d, d), lambda i: (0, 0)),
            pl.BlockSpec((1, d), lambda i: (0, 0)),
        ],
        out_specs=pl.BlockSpec((blk, d), lambda i: (i, 0)),
        out_shape=jax.ShapeDtypeStruct((out_rows, d), jnp.float32),
    )(parts, degt, w, b)


def kernel(in_feat, edge_index, W1, b1, W2, b2):
    n, d = in_feat.shape
    e = edge_index.shape[1]
    assert e % NW == 0
    epw = e // NW
    cpw = -(-epw // CHUNK)
    if cpw % 2:
        cpw += 1
    pw_pad = cpw * CHUNK - epw
    n_pad = -(-(n + max(pw_pad, 1)) // 1024) * 1024
    assert n_pad - n >= pw_pad and n_pad % 1024 == 0

    ei = edge_index.astype(jnp.int32)
    pads = jnp.broadcast_to(
        jnp.arange(pw_pad, dtype=jnp.int32) + n, (NW, pw_pad))
    srcs = jnp.concatenate([ei[0].reshape(NW, epw), pads],
                           axis=1).reshape(NW, cpw, CHUNK)
    dsts = jnp.concatenate([ei[1].reshape(NW, epw), pads],
                           axis=1).reshape(NW, cpw, CHUNK)
    x_p = jnp.pad(in_feat, ((0, n_pad - n), (0, 0)))

    deg = _make_deg_kernel(n_pad, cpw)(srcs, dsts)
    degt = deg.T

    spmm = _make_spmm_kernel(n_pad, d, cpw)

    h1 = _pre_call(x_p, degt, n_pad, d)
    parts1 = spmm(h1, srcs, dsts)
    h2 = _post_call(parts1, degt, W1, b1.reshape(1, d), mid_layer=True,
                    n_real=n, n_pad=n_pad, d=d, out_rows=n_pad, blk=2048)
    parts2 = spmm(h2, srcs, dsts)
    assert n % 2000 == 0
    return _post_call(parts2, degt, W2, b2.reshape(1, d), mid_layer=False,
                      n_real=n, n_pad=n_pad, d=d, out_rows=n, blk=2000)


# speedup vs baseline: 1.2369x; 1.2369x over previous
import functools

import jax
import jax.numpy as jnp
from jax import lax
from jax.experimental import pallas as pl
from jax.experimental.pallas import tpu as pltpu
from jax.experimental.pallas import tpu_sc as plsc

NC = 2
NS = 16
NW = NC * NS
LANES = 16

CHUNK = 128


def _sc_mesh():
    return plsc.VectorSubcoreMesh(
        core_axis_name="c", subcore_axis_name="s", num_cores=NC, num_subcores=NS
    )


def _make_deg_kernel(n_pad, cpw):
    nps = n_pad // NS

    def body(srcs, dsts, deg_out, sidx, didx, ones_v, zbuf, deg_s, deg_d,
             sem0):
        c = lax.axis_index("c")
        s = lax.axis_index("s")
        wid = c * NS + s

        z16 = jnp.zeros((LANES,), jnp.float32)
        o16 = jnp.ones((LANES,), jnp.float32)

        def zb(i, _):
            zbuf[pl.ds(i * LANES, LANES)] = z16
            return 0

        lax.fori_loop(0, nps // LANES, zb, 0)

        def ob(i, _):
            ones_v[pl.ds(i * LANES, LANES)] = o16
            return 0

        lax.fori_loop(0, CHUNK // LANES, ob, 0)

        pltpu.sync_copy(zbuf, deg_s.at[pl.ds(s * nps, nps)])
        pltpu.sync_copy(zbuf, deg_d.at[pl.ds(s * nps, nps)])
        pltpu.sync_copy(srcs.at[wid], sidx)
        pltpu.sync_copy(dsts.at[wid], didx)
        plsc.subcore_barrier()

        grp = 8

        def step(q, _):
            for k in range(grp):
                g = q * grp + k
                pltpu.async_copy(ones_v, deg_s.at[sidx.at[g]], sem0, add=True)
                pltpu.async_copy(ones_v, deg_d.at[didx.at[g]], sem0, add=True)
            for k in range(grp):
                g = q * grp + k
                pltpu.make_async_copy(ones_v, deg_s.at[sidx.at[g]],
                                      sem0).wait()
                pltpu.make_async_copy(ones_v, deg_d.at[didx.at[g]],
                                      sem0).wait()
            return 0

        assert cpw % grp == 0
        lax.fori_loop(0, cpw // grp, step, 0)
        plsc.subcore_barrier()

        pltpu.sync_copy(deg_s.at[pl.ds(s * nps, nps)],
                        deg_out.at[2 * c, pl.ds(s * nps, nps)])
        pltpu.sync_copy(deg_d.at[pl.ds(s * nps, nps)],
                        deg_out.at[2 * c + 1, pl.ds(s * nps, nps)])

    return pl.kernel(
        body,
        out_type=jax.ShapeDtypeStruct((2 * NC, n_pad), jnp.float32),
        mesh=_sc_mesh(),
        scratch_types=[
            pltpu.VMEM((cpw, CHUNK), jnp.int32),
            pltpu.VMEM((cpw, CHUNK), jnp.int32),
            pltpu.VMEM((CHUNK,), jnp.float32),
            pltpu.VMEM((nps,), jnp.float32),
            pltpu.VMEM_SHARED((n_pad,), jnp.float32),
            pltpu.VMEM_SHARED((n_pad,), jnp.float32),
            pltpu.SemaphoreType.DMA,
        ],
    )


def _make_spmm_kernel(n_pad, d, cpw):
    nps = n_pad // NS
    passes = 2
    assert cpw % passes == 0 and (cpw // passes) % 2 == 0
    hcw = cpw // passes
    assert nps % CHUNK == 0

    def body(h_hbm, srcs, dsts, out_hbm, sidx, didx, buf0, buf1,
             agg, sem0, sem1):
        c = lax.axis_index("c")
        s = lax.axis_index("s")
        wid = c * NS + s

        z16 = jnp.zeros((LANES,), jnp.float32)

        def zb(i, _):
            r = i // (d // LANES)
            k = i % (d // LANES)
            buf0[r, pl.ds(k * LANES, LANES)] = z16
            return 0

        lax.fori_loop(0, CHUNK * (d // LANES), zb, 0)

        def zagg(k, _):
            pltpu.sync_copy(buf0, agg.at[pl.ds(s * nps + k * CHUNK, CHUNK)])
            return 0

        lax.fori_loop(0, nps // CHUNK, zagg, 0)
        plsc.subcore_barrier()

        for p in range(passes):
            pltpu.sync_copy(srcs.at[wid, pl.ds(p * hcw, hcw)], sidx)
            pltpu.sync_copy(dsts.at[wid, pl.ds(p * hcw, hcw)], didx)

            pltpu.async_copy(h_hbm.at[sidx.at[0]], buf0, sem0)

            def step(g2, _):
                g = g2 * 2
                pltpu.async_copy(h_hbm.at[sidx.at[g + 1]], buf1, sem1)
                pltpu.make_async_copy(h_hbm.at[sidx.at[g]], buf0, sem0).wait()
                pltpu.sync_copy(buf0, agg.at[didx.at[g]], add=True)

                @pl.when(g + 2 < hcw)
                def _():
                    pltpu.async_copy(h_hbm.at[sidx.at[g + 2]], buf0, sem0)

                pltpu.make_async_copy(h_hbm.at[sidx.at[g + 1]], buf1,
                                      sem1).wait()
                pltpu.sync_copy(buf1, agg.at[didx.at[g + 1]], add=True)
                return 0

            lax.fori_loop(0, hcw // 2, step, 0)

        plsc.subcore_barrier()
        pltpu.sync_copy(agg.at[pl.ds(s * nps, nps)],
                        out_hbm.at[c, pl.ds(s * nps, nps)])

    return pl.kernel(
        body,
        out_type=jax.ShapeDtypeStruct((NC, n_pad, d), jnp.float32),
        mesh=_sc_mesh(),
        scratch_types=[
            pltpu.VMEM((hcw, CHUNK), jnp.int32),
            pltpu.VMEM((hcw, CHUNK), jnp.int32),
            pltpu.VMEM((CHUNK, d), jnp.float32),
            pltpu.VMEM((CHUNK, d), jnp.float32),
            pltpu.VMEM_SHARED((n_pad, d), jnp.float32),
            pltpu.SemaphoreType.DMA,
            pltpu.SemaphoreType.DMA,
        ],
    )


def _norm_from(deg):
    return jnp.where(deg > 0, lax.rsqrt(jnp.maximum(deg, 1.0)), 0.0)


def _pre_body(x_ref, degt_ref, h_ref):
    deg_src = degt_ref[:, 0:1] + degt_ref[:, 2:3]
    h_ref[...] = x_ref[...] * _norm_from(deg_src)


def _post_body(parts_ref, degt_ref, w_ref, b_ref, out_ref, *, mid_layer):
    agg = parts_ref[0] + parts_ref[1]
    deg_dst = degt_ref[:, 1:2] + degt_ref[:, 3:4]
    z = jnp.dot(agg * _norm_from(deg_dst), w_ref[...],
                preferred_element_type=jnp.float32) + b_ref[...]
    if mid_layer:
        z = jnp.maximum(z, 0.0)
        deg_src = degt_ref[:, 0:1] + degt_ref[:, 2:3]
        z = z * _norm_from(deg_src)
    out_ref[...] = z


def _pre_call(x, degt, n_rows, d, blk=2000):
    grid = (n_rows // blk,)
    return pl.pallas_call(
        _pre_body,
        grid=grid,
        in_specs=[
            pl.BlockSpec((blk, d), lambda i: (i, 0)),
            pl.BlockSpec((blk, 2 * NC), lambda i: (i, 0)),
        ],
        out_specs=pl.BlockSpec((blk, d), lambda i: (i, 0)),
        out_shape=jax.ShapeDtypeStruct((n_rows, d), jnp.float32),
    )(x, degt)


def _post_call(parts, degt, w, b, *, mid_layer, d, out_rows, blk):
    grid = (out_rows // blk,)
    body = functools.partial(_post_body, mid_layer=mid_layer)
    return pl.pallas_call(
        body,
        grid=grid,
        in_specs=[
            pl.BlockSpec((NC, blk, d), lambda i: (0, i, 0)),
            pl.BlockSpec((blk, 2 * NC), lambda i: (i, 0)),
            pl.BlockSpec((d, d), lambda i: (0, 0)),
            pl.BlockSpec((1, d), lambda i: (0, 0)),
        ],
        out_specs=pl.BlockSpec((blk, d), lambda i: (i, 0)),
        out_shape=jax.ShapeDtypeStruct((out_rows, d), jnp.float32),
    )(parts, degt, w, b)


def kernel(in_feat, edge_index, W1, b1, W2, b2):
    n, d = in_feat.shape
    e = edge_index.shape[1]
    assert e % NW == 0
    epw = e // NW
    cpw = -(-epw // CHUNK)
    if cpw % 2:
        cpw += 1
    pw_pad = cpw * CHUNK - epw
    n_pad = -(-(n + max(pw_pad, 1)) // 1024) * 1024
    assert n_pad - n >= pw_pad and n_pad % 1024 == 0

    ei = edge_index.astype(jnp.int32)
    pad_ids = jnp.arange(pw_pad, dtype=jnp.int32)
    pads_hi = jnp.broadcast_to(pad_ids % (n_pad - n) + n, (NW, pw_pad))
    pads_lo = jnp.broadcast_to(pad_ids % n, (NW, pw_pad))
    srcs_deg = jnp.concatenate([ei[0].reshape(NW, epw), pads_hi],
                               axis=1).reshape(NW, cpw, CHUNK)
    srcs = jnp.concatenate([ei[0].reshape(NW, epw), pads_lo],
                           axis=1).reshape(NW, cpw, CHUNK)
    dsts = jnp.concatenate([ei[1].reshape(NW, epw), pads_hi],
                           axis=1).reshape(NW, cpw, CHUNK)

    deg = _make_deg_kernel(n_pad, cpw)(srcs_deg, dsts)
    degt = deg.T

    spmm = _make_spmm_kernel(n_pad, d, cpw)

    assert n % 2000 == 0
    h1 = _pre_call(in_feat, degt, n, d)
    parts1 = spmm(h1, srcs, dsts)
    h2 = _post_call(parts1, degt, W1, b1.reshape(1, d), mid_layer=True,
                    d=d, out_rows=n, blk=2000)
    parts2 = spmm(h2, srcs, dsts)
    return _post_call(parts2, degt, W2, b2.reshape(1, d), mid_layer=False,
                      d=d, out_rows=n, blk=2000)


# speedup vs baseline: 1.2803x; 1.0350x over previous
import functools

import jax
import jax.numpy as jnp
from jax import lax
from jax.experimental import pallas as pl
from jax.experimental.pallas import tpu as pltpu
from jax.experimental.pallas import tpu_sc as plsc

NC = 2
NS = 16
NW = NC * NS
LANES = 16

CHUNK = 128
PC = 40


def _sc_mesh():
    return plsc.VectorSubcoreMesh(
        core_axis_name="c", subcore_axis_name="s", num_cores=NC, num_subcores=NS
    )


def _chunk_split(total_chunks):
    cap = 2 * PC
    nfull = total_chunks // cap
    rem = total_chunks - nfull * cap
    assert nfull <= NW and (nfull < NW or rem == 0)
    assert rem % 2 == 0 and cap % 8 == 0
    return cap, nfull, rem


def _make_deg_kernel(n, total_chunks):
    cap, nfull, rem = _chunk_split(total_chunks)
    n_tbl = -(-n // (NS * LANES)) * (NS * LANES)
    nps = n_tbl // NS
    grp = 4
    assert cap % grp == 0 and rem % grp == 0

    def body(srcs2, dsts2, deg_out, sidx, didx, ones_v, zbuf, deg_s, deg_d,
             sem0):
        c = lax.axis_index("c")
        s = lax.axis_index("s")
        wid = c * NS + s

        z16 = jnp.zeros((LANES,), jnp.float32)
        o16 = jnp.ones((LANES,), jnp.float32)

        def zb(i, _):
            zbuf[pl.ds(i * LANES, LANES)] = z16
            return 0

        lax.fori_loop(0, nps // LANES, zb, 0)
        for k in range(CHUNK // LANES):
            ones_v[pl.ds(k * LANES, LANES)] = o16

        pltpu.sync_copy(zbuf, deg_s.at[pl.ds(s * nps, nps)])
        pltpu.sync_copy(zbuf, deg_d.at[pl.ds(s * nps, nps)])

        def run(base, cnt):
            pltpu.sync_copy(srcs2.at[pl.ds(base, cnt)],
                            sidx.at[pl.ds(0, cnt)])
            pltpu.sync_copy(dsts2.at[pl.ds(base, cnt)],
                            didx.at[pl.ds(0, cnt)])

            def step(q, _):
                for k in range(grp):
                    g = q * grp + k
                    pltpu.async_copy(ones_v, deg_s.at[sidx.at[g]], sem0,
                                     add=True)
                    pltpu.async_copy(ones_v, deg_d.at[didx.at[g]], sem0,
                                     add=True)
                for k in range(grp):
                    g = q * grp + k
                    pltpu.make_async_copy(ones_v, deg_s.at[sidx.at[g]],
                                          sem0).wait()
                    pltpu.make_async_copy(ones_v, deg_d.at[didx.at[g]],
                                          sem0).wait()
                return 0

            lax.fori_loop(0, cnt // grp, step, 0)

        plsc.subcore_barrier()

        @pl.when(wid < nfull)
        def _():
            run(cap * wid, cap)

        if rem:
            @pl.when(wid == nfull)
            def _():
                run(cap * nfull, rem)

        plsc.subcore_barrier()

        pltpu.sync_copy(deg_s.at[pl.ds(s * nps, nps)],
                        deg_out.at[2 * c, pl.ds(s * nps, nps)])
        pltpu.sync_copy(deg_d.at[pl.ds(s * nps, nps)],
                        deg_out.at[2 * c + 1, pl.ds(s * nps, nps)])

    return pl.kernel(
        body,
        out_type=jax.ShapeDtypeStruct((2 * NC, n_tbl), jnp.float32),
        mesh=_sc_mesh(),
        scratch_types=[
            pltpu.VMEM((cap, CHUNK), jnp.int32),
            pltpu.VMEM((cap, CHUNK), jnp.int32),
            pltpu.VMEM((CHUNK,), jnp.float32),
            pltpu.VMEM((nps,), jnp.float32),
            pltpu.VMEM_SHARED((n_tbl,), jnp.float32),
            pltpu.VMEM_SHARED((n_tbl,), jnp.float32),
            pltpu.SemaphoreType.DMA,
        ],
    ), n_tbl


def _make_spmm_kernel(n, d, total_chunks):
    cap, nfull, rem = _chunk_split(total_chunks)
    n_sp = -(-n // (NS * CHUNK)) * (NS * CHUNK)
    nps = n_sp // NS
    zfull, zrem = divmod(nps, CHUNK)

    def body(h_hbm, srcs2, dsts2, out_hbm, sidx, didx, buf0, buf1,
             agg, sem0, sem1):
        c = lax.axis_index("c")
        s = lax.axis_index("s")
        wid = c * NS + s

        z16 = jnp.zeros((LANES,), jnp.float32)

        def zb(i, _):
            for k in range(d // LANES):
                buf0[i, pl.ds(k * LANES, LANES)] = z16
            return 0

        lax.fori_loop(0, CHUNK, zb, 0)
        for k in range(zfull):
            pltpu.sync_copy(buf0, agg.at[pl.ds(s * nps + k * CHUNK, CHUNK)])
        if zrem:
            pltpu.sync_copy(buf0.at[pl.ds(0, zrem)],
                            agg.at[pl.ds(s * nps + zfull * CHUNK, zrem)])
        plsc.subcore_barrier()

        def run(base, pk):
            pltpu.sync_copy(srcs2.at[pl.ds(base, pk)], sidx.at[pl.ds(0, pk)])
            pltpu.sync_copy(dsts2.at[pl.ds(base, pk)], didx.at[pl.ds(0, pk)])

            pltpu.async_copy(h_hbm.at[sidx.at[0]], buf0, sem0)

            def step(g2, _):
                g = g2 * 2
                pltpu.async_copy(h_hbm.at[sidx.at[g + 1]], buf1, sem1)
                pltpu.make_async_copy(h_hbm.at[sidx.at[g]], buf0, sem0).wait()
                pltpu.sync_copy(buf0, agg.at[didx.at[g]], add=True)

                @pl.when(g + 2 < pk)
                def _():
                    pltpu.async_copy(h_hbm.at[sidx.at[g + 2]], buf0, sem0)

                pltpu.make_async_copy(h_hbm.at[sidx.at[g + 1]], buf1,
                                      sem1).wait()
                pltpu.sync_copy(buf1, agg.at[didx.at[g + 1]], add=True)
                return 0

            lax.fori_loop(0, pk // 2, step, 0)

        @pl.when(wid < nfull)
        def _():
            run(cap * wid, PC)
            run(cap * wid + PC, cap - PC)

        if rem:
            @pl.when(wid == nfull)
            def _():
                run(cap * nfull, rem)

        plsc.subcore_barrier()
        pltpu.sync_copy(agg.at[pl.ds(s * nps, nps)],
                        out_hbm.at[c, pl.ds(s * nps, nps)])

    return pl.kernel(
        body,
        out_type=jax.ShapeDtypeStruct((NC, n_sp, d), jnp.float32),
        mesh=_sc_mesh(),
        scratch_types=[
            pltpu.VMEM((PC, CHUNK), jnp.int32),
            pltpu.VMEM((PC, CHUNK), jnp.int32),
            pltpu.VMEM((CHUNK, d), jnp.float32),
            pltpu.VMEM((CHUNK, d), jnp.float32),
            pltpu.VMEM_SHARED((n_sp, d), jnp.float32),
            pltpu.SemaphoreType.DMA,
            pltpu.SemaphoreType.DMA,
        ],
    )


def _norm_from(deg):
    return jnp.where(deg > 0, lax.rsqrt(jnp.maximum(deg, 1.0)), 0.0)


def _pre_body(x_ref, degt_ref, h_ref):
    deg_src = degt_ref[:, 0:1] + degt_ref[:, 2:3]
    h_ref[...] = x_ref[...] * _norm_from(deg_src)


def _post_body(parts_ref, degt_ref, w_ref, b_ref, out_ref, *, mid_layer):
    agg = parts_ref[0] + parts_ref[1]
    deg_dst = degt_ref[:, 1:2] + degt_ref[:, 3:4]
    z = jnp.dot(agg * _norm_from(deg_dst), w_ref[...],
                preferred_element_type=jnp.float32) + b_ref[...]
    if mid_layer:
        z = jnp.maximum(z, 0.0)
        deg_src = degt_ref[:, 0:1] + degt_ref[:, 2:3]
        z = z * _norm_from(deg_src)
    out_ref[...] = z


def _pre_call(x, degt, n_rows, d, blk=2000):
    grid = (n_rows // blk,)
    return pl.pallas_call(
        _pre_body,
        grid=grid,
        in_specs=[
            pl.BlockSpec((blk, d), lambda i: (i, 0)),
            pl.BlockSpec((blk, 2 * NC), lambda i: (i, 0)),
        ],
        out_specs=pl.BlockSpec((blk, d), lambda i: (i, 0)),
        out_shape=jax.ShapeDtypeStruct((n_rows, d), jnp.float32),
    )(x, degt)


def _post_call(parts, degt, w, b, *, mid_layer, d, out_rows, blk):
    grid = (out_rows // blk,)
    body = functools.partial(_post_body, mid_layer=mid_layer)
    return pl.pallas_call(
        body,
        grid=grid,
        in_specs=[
            pl.BlockSpec((NC, blk, d), lambda i: (0, i, 0)),
            pl.BlockSpec((blk, 2 * NC), lambda i: (i, 0)),
            pl.BlockSpec((d, d), lambda i: (0, 0)),
            pl.BlockSpec((1, d), lambda i: (0, 0)),
        ],
        out_specs=pl.BlockSpec((blk, d), lambda i: (i, 0)),
        out_shape=jax.ShapeDtypeStruct((out_rows, d), jnp.float32),
    )(parts, degt, w, b)


def kernel(in_feat, edge_index, W1, b1, W2, b2):
    n, d = in_feat.shape
    e = edge_index.shape[1]
    assert e % CHUNK == 0 and n % NS == 0
    total_chunks = e // CHUNK

    ei = edge_index.astype(jnp.int32)
    srcs2 = ei[0].reshape(total_chunks, CHUNK)
    dsts2 = ei[1].reshape(total_chunks, CHUNK)

    deg_kernel, _ = _make_deg_kernel(n, total_chunks)
    deg = deg_kernel(srcs2, dsts2)
    degt = deg.T

    spmm = _make_spmm_kernel(n, d, total_chunks)

    assert n % 2000 == 0
    h1 = _pre_call(in_feat, degt, n, d)
    parts1 = spmm(h1, srcs2, dsts2)
    h2 = _post_call(parts1, degt, W1, b1.reshape(1, d), mid_layer=True,
                    d=d, out_rows=n, blk=2000)
    parts2 = spmm(h2, srcs2, dsts2)
    return _post_call(parts2, degt, W2, b2.reshape(1, d), mid_layer=False,
                      d=d, out_rows=n, blk=2000)


# speedup vs baseline: 1.3213x; 1.0321x over previous
import functools

import jax
import jax.numpy as jnp
from jax import lax
from jax.experimental import pallas as pl
from jax.experimental.pallas import tpu as pltpu
from jax.experimental.pallas import tpu_sc as plsc

NC = 2
NS = 16
NW = NC * NS
LANES = 16

CHUNK = 128
PC = 40


def _sc_mesh():
    return plsc.VectorSubcoreMesh(
        core_axis_name="c", subcore_axis_name="s", num_cores=NC, num_subcores=NS
    )


def _chunk_split(total_chunks):
    cap = 2 * PC
    nfull = total_chunks // cap
    rem = total_chunks - nfull * cap
    assert nfull <= NW and (nfull < NW or rem == 0)
    assert rem % 2 == 0 and cap % 8 == 0
    return cap, nfull, rem


def _make_deg_kernel(n, total_chunks):
    cap, nfull, rem = _chunk_split(total_chunks)
    n_tbl = -(-n // (NS * LANES)) * (NS * LANES)
    nps = n_tbl // NS
    grp = 4
    assert cap % grp == 0 and rem % grp == 0

    def body(srcs2, dsts2, deg_out, sidx, didx, ones_v, zbuf, deg_s, deg_d,
             sem0):
        c = lax.axis_index("c")
        s = lax.axis_index("s")
        wid = c * NS + s

        z16 = jnp.zeros((LANES,), jnp.float32)
        o16 = jnp.ones((LANES,), jnp.float32)

        def zb(i, _):
            zbuf[pl.ds(i * LANES, LANES)] = z16
            return 0

        lax.fori_loop(0, nps // LANES, zb, 0)
        for k in range(CHUNK // LANES):
            ones_v[pl.ds(k * LANES, LANES)] = o16

        pltpu.sync_copy(zbuf, deg_s.at[pl.ds(s * nps, nps)])
        pltpu.sync_copy(zbuf, deg_d.at[pl.ds(s * nps, nps)])

        def run(base, cnt):
            pltpu.sync_copy(srcs2.at[pl.ds(base, cnt)],
                            sidx.at[pl.ds(0, cnt)])
            pltpu.sync_copy(dsts2.at[pl.ds(base, cnt)],
                            didx.at[pl.ds(0, cnt)])

            def step(q, _):
                for k in range(grp):
                    g = q * grp + k
                    pltpu.async_copy(ones_v, deg_s.at[sidx.at[g]], sem0,
                                     add=True)
                    pltpu.async_copy(ones_v, deg_d.at[didx.at[g]], sem0,
                                     add=True)
                for k in range(grp):
                    g = q * grp + k
                    pltpu.make_async_copy(ones_v, deg_s.at[sidx.at[g]],
                                          sem0).wait()
                    pltpu.make_async_copy(ones_v, deg_d.at[didx.at[g]],
                                          sem0).wait()
                return 0

            lax.fori_loop(0, cnt // grp, step, 0)

        plsc.subcore_barrier()

        @pl.when(wid < nfull)
        def _():
            run(cap * wid, cap)

        if rem:
            @pl.when(wid == nfull)
            def _():
                run(cap * nfull, rem)

        plsc.subcore_barrier()

        pltpu.sync_copy(deg_s.at[pl.ds(s * nps, nps)],
                        deg_out.at[2 * c, pl.ds(s * nps, nps)])
        pltpu.sync_copy(deg_d.at[pl.ds(s * nps, nps)],
                        deg_out.at[2 * c + 1, pl.ds(s * nps, nps)])

    return pl.kernel(
        body,
        out_type=jax.ShapeDtypeStruct((2 * NC, n_tbl), jnp.float32),
        mesh=_sc_mesh(),
        scratch_types=[
            pltpu.VMEM((cap, CHUNK), jnp.int32),
            pltpu.VMEM((cap, CHUNK), jnp.int32),
            pltpu.VMEM((CHUNK,), jnp.float32),
            pltpu.VMEM((nps,), jnp.float32),
            pltpu.VMEM_SHARED((n_tbl,), jnp.float32),
            pltpu.VMEM_SHARED((n_tbl,), jnp.float32),
            pltpu.SemaphoreType.DMA,
        ],
    ), n_tbl


def _make_spmm_kernel(n, d, total_chunks):
    cap, nfull, rem = _chunk_split(total_chunks)
    n_sp = -(-n // (NS * CHUNK)) * (NS * CHUNK)
    nps = n_sp // NS
    zfull, zrem = divmod(nps, CHUNK)

    def body(h_hbm, srcs2, dsts2, out_hbm, sidx, didx, buf0, buf1,
             agg, sem0, sem1):
        c = lax.axis_index("c")
        s = lax.axis_index("s")
        wid = c * NS + s

        z16 = jnp.zeros((LANES,), jnp.float32)

        def zb(i, _):
            for k in range(d // LANES):
                buf0[i, pl.ds(k * LANES, LANES)] = z16
            return 0

        lax.fori_loop(0, CHUNK, zb, 0)
        for k in range(zfull):
            pltpu.sync_copy(buf0, agg.at[pl.ds(s * nps + k * CHUNK, CHUNK)])
        if zrem:
            pltpu.sync_copy(buf0.at[pl.ds(0, zrem)],
                            agg.at[pl.ds(s * nps + zfull * CHUNK, zrem)])
        plsc.subcore_barrier()

        def run(base, pk):
            pltpu.sync_copy(srcs2.at[pl.ds(base, pk)], sidx.at[pl.ds(0, pk)])
            pltpu.sync_copy(dsts2.at[pl.ds(base, pk)], didx.at[pl.ds(0, pk)])

            pltpu.async_copy(h_hbm.at[sidx.at[0]], buf0, sem0)

            def step(g2, _):
                g = g2 * 2
                pltpu.async_copy(h_hbm.at[sidx.at[g + 1]], buf1, sem1)
                pltpu.make_async_copy(h_hbm.at[sidx.at[g]], buf0, sem0).wait()
                pltpu.sync_copy(buf0, agg.at[didx.at[g]], add=True)

                @pl.when(g + 2 < pk)
                def _():
                    pltpu.async_copy(h_hbm.at[sidx.at[g + 2]], buf0, sem0)

                pltpu.make_async_copy(h_hbm.at[sidx.at[g + 1]], buf1,
                                      sem1).wait()
                pltpu.sync_copy(buf1, agg.at[didx.at[g + 1]], add=True)
                return 0

            lax.fori_loop(0, pk // 2, step, 0)

        @pl.when(wid < nfull)
        def _():
            run(cap * wid, PC)
            run(cap * wid + PC, cap - PC)

        if rem:
            @pl.when(wid == nfull)
            def _():
                run(cap * nfull, rem)

        plsc.subcore_barrier()
        pltpu.sync_copy(agg.at[pl.ds(s * nps, nps)],
                        out_hbm.at[c, pl.ds(s * nps, nps)])

    return pl.kernel(
        body,
        out_type=jax.ShapeDtypeStruct((NC, n_sp, d), jnp.float32),
        mesh=_sc_mesh(),
        scratch_types=[
            pltpu.VMEM((PC, CHUNK), jnp.int32),
            pltpu.VMEM((PC, CHUNK), jnp.int32),
            pltpu.VMEM((CHUNK, d), jnp.float32),
            pltpu.VMEM((CHUNK, d), jnp.float32),
            pltpu.VMEM_SHARED((n_sp, d), jnp.float32),
            pltpu.SemaphoreType.DMA,
            pltpu.SemaphoreType.DMA,
        ],
    )


def _chunkify(ei, total_chunks):
    def body(ei_ref, s_ref, d_ref):
        s_ref[...] = ei_ref[0].reshape(total_chunks, CHUNK)
        d_ref[...] = ei_ref[1].reshape(total_chunks, CHUNK)

    return pl.pallas_call(
        body,
        out_shape=[
            jax.ShapeDtypeStruct((total_chunks, CHUNK), jnp.int32),
            jax.ShapeDtypeStruct((total_chunks, CHUNK), jnp.int32),
        ],
    )(ei)


def _norm_from(deg):
    return jnp.where(deg > 0, lax.rsqrt(jnp.maximum(deg, 1.0)), 0.0)


def _pre_body(x_ref, degt_ref, h_ref):
    deg_src = degt_ref[:, 0:1] + degt_ref[:, 2:3]
    h_ref[...] = x_ref[...] * _norm_from(deg_src)


def _post_body(parts_ref, degt_ref, w_ref, b_ref, out_ref, *, mid_layer):
    agg = parts_ref[0] + parts_ref[1]
    deg_dst = degt_ref[:, 1:2] + degt_ref[:, 3:4]
    z = jnp.dot(agg * _norm_from(deg_dst), w_ref[...],
                preferred_element_type=jnp.float32) + b_ref[...]
    if mid_layer:
        z = jnp.maximum(z, 0.0)
        deg_src = degt_ref[:, 0:1] + degt_ref[:, 2:3]
        z = z * _norm_from(deg_src)
    out_ref[...] = z


def _pre_call(x, degt, n_rows, d, blk=2000):
    grid = (n_rows // blk,)
    return pl.pallas_call(
        _pre_body,
        grid=grid,
        in_specs=[
            pl.BlockSpec((blk, d), lambda i: (i, 0)),
            pl.BlockSpec((blk, 2 * NC), lambda i: (i, 0)),
        ],
        out_specs=pl.BlockSpec((blk, d), lambda i: (i, 0)),
        out_shape=jax.ShapeDtypeStruct((n_rows, d), jnp.float32),
    )(x, degt)


def _post_call(parts, degt, w, b, *, mid_layer, d, out_rows, blk):
    grid = (out_rows // blk,)
    body = functools.partial(_post_body, mid_layer=mid_layer)
    return pl.pallas_call(
        body,
        grid=grid,
        in_specs=[
            pl.BlockSpec((NC, blk, d), lambda i: (0, i, 0)),
            pl.BlockSpec((blk, 2 * NC), lambda i: (i, 0)),
            pl.BlockSpec((d, d), lambda i: (0, 0)),
            pl.BlockSpec((1, d), lambda i: (0, 0)),
        ],
        out_specs=pl.BlockSpec((blk, d), lambda i: (i, 0)),
        out_shape=jax.ShapeDtypeStruct((out_rows, d), jnp.float32),
    )(parts, degt, w, b)


def kernel(in_feat, edge_index, W1, b1, W2, b2):
    n, d = in_feat.shape
    e = edge_index.shape[1]
    assert e % CHUNK == 0 and n % NS == 0
    total_chunks = e // CHUNK

    ei = edge_index.astype(jnp.int32)
    srcs2, dsts2 = _chunkify(ei, total_chunks)

    deg_kernel, _ = _make_deg_kernel(n, total_chunks)
    deg = deg_kernel(srcs2, dsts2)
    degt = deg.T

    spmm = _make_spmm_kernel(n, d, total_chunks)

    assert n % 2000 == 0
    h1 = _pre_call(in_feat, degt, n, d)
    parts1 = spmm(h1, srcs2, dsts2)
    h2 = _post_call(parts1, degt, W1, b1.reshape(1, d), mid_layer=True,
                    d=d, out_rows=n, blk=2000)
    parts2 = spmm(h2, srcs2, dsts2)
    return _post_call(parts2, degt, W2, b2.reshape(1, d), mid_layer=False,
                      d=d, out_rows=n, blk=2000)


# speedup vs baseline: 1.3538x; 1.0246x over previous
import functools

import jax
import jax.numpy as jnp
from jax import lax
from jax.experimental import pallas as pl
from jax.experimental.pallas import tpu as pltpu
from jax.experimental.pallas import tpu_sc as plsc

NC = 2
NS = 16
NW = NC * NS
LANES = 16

CHUNK = 128
PC = 40


def _sc_mesh():
    return plsc.VectorSubcoreMesh(
        core_axis_name="c", subcore_axis_name="s", num_cores=NC, num_subcores=NS
    )


def _chunk_split(total_chunks):
    cap = 2 * PC
    nfull = total_chunks // cap
    rem = total_chunks - nfull * cap
    assert nfull <= NW and (nfull < NW or rem == 0)
    assert rem % 2 == 0 and cap % 8 == 0
    return cap, nfull, rem


def _make_deg_kernel(n, total_chunks):
    cap, nfull, rem = _chunk_split(total_chunks)
    n_tbl = -(-n // (NS * LANES)) * (NS * LANES)
    nps = n_tbl // NS
    grp_full = 8
    grp_rem = 4
    assert cap % grp_full == 0 and rem % grp_rem == 0

    def body(srcs2, dsts2, deg_out, sidx, didx, ones_v, zbuf, deg_s, deg_d,
             sem0):
        c = lax.axis_index("c")
        s = lax.axis_index("s")
        wid = c * NS + s

        z16 = jnp.zeros((LANES,), jnp.float32)
        o16 = jnp.ones((LANES,), jnp.float32)

        def zb(i, _):
            zbuf[pl.ds(i * LANES, LANES)] = z16
            return 0

        lax.fori_loop(0, nps // LANES, zb, 0)
        for k in range(CHUNK // LANES):
            ones_v[pl.ds(k * LANES, LANES)] = o16

        pltpu.sync_copy(zbuf, deg_s.at[pl.ds(s * nps, nps)])
        pltpu.sync_copy(zbuf, deg_d.at[pl.ds(s * nps, nps)])

        def run(base, cnt, grp):
            pltpu.sync_copy(srcs2.at[pl.ds(base, cnt)],
                            sidx.at[pl.ds(0, cnt)])
            pltpu.sync_copy(dsts2.at[pl.ds(base, cnt)],
                            didx.at[pl.ds(0, cnt)])

            def step(q, _):
                for k in range(grp):
                    g = q * grp + k
                    pltpu.async_copy(ones_v, deg_s.at[sidx.at[g]], sem0,
                                     add=True)
                    pltpu.async_copy(ones_v, deg_d.at[didx.at[g]], sem0,
                                     add=True)
                for k in range(grp):
                    g = q * grp + k
                    pltpu.make_async_copy(ones_v, deg_s.at[sidx.at[g]],
                                          sem0).wait()
                    pltpu.make_async_copy(ones_v, deg_d.at[didx.at[g]],
                                          sem0).wait()
                return 0

            lax.fori_loop(0, cnt // grp, step, 0)

        plsc.subcore_barrier()

        @pl.when(wid < nfull)
        def _():
            run(cap * wid, cap, grp_full)

        if rem:
            @pl.when(wid == nfull)
            def _():
                run(cap * nfull, rem, grp_rem)

        plsc.subcore_barrier()

        pltpu.sync_copy(deg_s.at[pl.ds(s * nps, nps)],
                        deg_out.at[2 * c, pl.ds(s * nps, nps)])
        pltpu.sync_copy(deg_d.at[pl.ds(s * nps, nps)],
                        deg_out.at[2 * c + 1, pl.ds(s * nps, nps)])

    return pl.kernel(
        body,
        out_type=jax.ShapeDtypeStruct((2 * NC, n_tbl), jnp.float32),
        mesh=_sc_mesh(),
        scratch_types=[
            pltpu.VMEM((cap, CHUNK), jnp.int32),
            pltpu.VMEM((cap, CHUNK), jnp.int32),
            pltpu.VMEM((CHUNK,), jnp.float32),
            pltpu.VMEM((nps,), jnp.float32),
            pltpu.VMEM_SHARED((n_tbl,), jnp.float32),
            pltpu.VMEM_SHARED((n_tbl,), jnp.float32),
            pltpu.SemaphoreType.DMA,
        ],
    ), n_tbl


def _make_spmm_kernel(n, d, total_chunks):
    cap, nfull, rem = _chunk_split(total_chunks)
    n_sp = -(-n // (NS * CHUNK)) * (NS * CHUNK)
    nps = n_sp // NS
    zfull, zrem = divmod(nps, CHUNK)

    def body(h_hbm, srcs2, dsts2, out_hbm, sidx, didx, buf0, buf1,
             agg, sem0, sem1):
        c = lax.axis_index("c")
        s = lax.axis_index("s")
        wid = c * NS + s

        z16 = jnp.zeros((LANES,), jnp.float32)

        def zb(i, _):
            for k in range(d // LANES):
                buf0[i, pl.ds(k * LANES, LANES)] = z16
            return 0

        lax.fori_loop(0, CHUNK, zb, 0)
        for k in range(zfull):
            pltpu.sync_copy(buf0, agg.at[pl.ds(s * nps + k * CHUNK, CHUNK)])
        if zrem:
            pltpu.sync_copy(buf0.at[pl.ds(0, zrem)],
                            agg.at[pl.ds(s * nps + zfull * CHUNK, zrem)])
        plsc.subcore_barrier()

        def run(base, pk):
            pltpu.sync_copy(srcs2.at[pl.ds(base, pk)], sidx.at[pl.ds(0, pk)])
            pltpu.sync_copy(dsts2.at[pl.ds(base, pk)], didx.at[pl.ds(0, pk)])

            pltpu.async_copy(h_hbm.at[sidx.at[0]], buf0, sem0)

            def step(g2, _):
                g = g2 * 2
                pltpu.async_copy(h_hbm.at[sidx.at[g + 1]], buf1, sem1)
                pltpu.make_async_copy(h_hbm.at[sidx.at[g]], buf0, sem0).wait()
                pltpu.sync_copy(buf0, agg.at[didx.at[g]], add=True)

                @pl.when(g + 2 < pk)
                def _():
                    pltpu.async_copy(h_hbm.at[sidx.at[g + 2]], buf0, sem0)

                pltpu.make_async_copy(h_hbm.at[sidx.at[g + 1]], buf1,
                                      sem1).wait()
                pltpu.sync_copy(buf1, agg.at[didx.at[g + 1]], add=True)
                return 0

            lax.fori_loop(0, pk // 2, step, 0)

        @pl.when(wid < nfull)
        def _():
            run(cap * wid, PC)
            run(cap * wid + PC, cap - PC)

        if rem:
            @pl.when(wid == nfull)
            def _():
                run(cap * nfull, rem)

        plsc.subcore_barrier()
        pltpu.sync_copy(agg.at[pl.ds(s * nps, nps)],
                        out_hbm.at[c, pl.ds(s * nps, nps)])

    return pl.kernel(
        body,
        out_type=jax.ShapeDtypeStruct((NC, n_sp, d), jnp.float32),
        mesh=_sc_mesh(),
        scratch_types=[
            pltpu.VMEM((PC, CHUNK), jnp.int32),
            pltpu.VMEM((PC, CHUNK), jnp.int32),
            pltpu.VMEM((CHUNK, d), jnp.float32),
            pltpu.VMEM((CHUNK, d), jnp.float32),
            pltpu.VMEM_SHARED((n_sp, d), jnp.float32),
            pltpu.SemaphoreType.DMA,
            pltpu.SemaphoreType.DMA,
        ],
    )


def _chunkify(ei, total_chunks):
    def body(ei_ref, s_ref, d_ref):
        s_ref[...] = ei_ref[0].reshape(total_chunks, CHUNK)
        d_ref[...] = ei_ref[1].reshape(total_chunks, CHUNK)

    return pl.pallas_call(
        body,
        out_shape=[
            jax.ShapeDtypeStruct((total_chunks, CHUNK), jnp.int32),
            jax.ShapeDtypeStruct((total_chunks, CHUNK), jnp.int32),
        ],
    )(ei)


def _norm_from(deg):
    return jnp.where(deg > 0, lax.rsqrt(jnp.maximum(deg, 1.0)), 0.0)


def _pre_body(x_ref, degt_ref, h_ref):
    deg_src = degt_ref[:, 0:1] + degt_ref[:, 2:3]
    h_ref[...] = x_ref[...] * _norm_from(deg_src)


def _post_body(parts_ref, degt_ref, w_ref, b_ref, out_ref, *, mid_layer):
    agg = parts_ref[0] + parts_ref[1]
    deg_dst = degt_ref[:, 1:2] + degt_ref[:, 3:4]
    z = jnp.dot(agg * _norm_from(deg_dst), w_ref[...],
                preferred_element_type=jnp.float32) + b_ref[...]
    if mid_layer:
        z = jnp.maximum(z, 0.0)
        deg_src = degt_ref[:, 0:1] + degt_ref[:, 2:3]
        z = z * _norm_from(deg_src)
    out_ref[...] = z


def _pre_call(x, degt, n_rows, d, blk=2000):
    grid = (n_rows // blk,)
    return pl.pallas_call(
        _pre_body,
        grid=grid,
        in_specs=[
            pl.BlockSpec((blk, d), lambda i: (i, 0)),
            pl.BlockSpec((blk, 2 * NC), lambda i: (i, 0)),
        ],
        out_specs=pl.BlockSpec((blk, d), lambda i: (i, 0)),
        out_shape=jax.ShapeDtypeStruct((n_rows, d), jnp.float32),
    )(x, degt)


def _post_call(parts, degt, w, b, *, mid_layer, d, out_rows, blk):
    grid = (out_rows // blk,)
    body = functools.partial(_post_body, mid_layer=mid_layer)
    return pl.pallas_call(
        body,
        grid=grid,
        in_specs=[
            pl.BlockSpec((NC, blk, d), lambda i: (0, i, 0)),
            pl.BlockSpec((blk, 2 * NC), lambda i: (i, 0)),
            pl.BlockSpec((d, d), lambda i: (0, 0)),
            pl.BlockSpec((1, d), lambda i: (0, 0)),
        ],
        out_specs=pl.BlockSpec((blk, d), lambda i: (i, 0)),
        out_shape=jax.ShapeDtypeStruct((out_rows, d), jnp.float32),
    )(parts, degt, w, b)


def kernel(in_feat, edge_index, W1, b1, W2, b2):
    n, d = in_feat.shape
    e = edge_index.shape[1]
    assert e % CHUNK == 0 and n % NS == 0
    total_chunks = e // CHUNK

    ei = edge_index.astype(jnp.int32)
    srcs2, dsts2 = _chunkify(ei, total_chunks)

    deg_kernel, _ = _make_deg_kernel(n, total_chunks)
    deg = deg_kernel(srcs2, dsts2)
    degt = deg.T

    spmm = _make_spmm_kernel(n, d, total_chunks)

    assert n % 2000 == 0
    h1 = _pre_call(in_feat, degt, n, d, blk=5000)
    parts1 = spmm(h1, srcs2, dsts2)
    h2 = _post_call(parts1, degt, W1, b1.reshape(1, d), mid_layer=True,
                    d=d, out_rows=n, blk=5000)
    parts2 = spmm(h2, srcs2, dsts2)
    return _post_call(parts2, degt, W2, b2.reshape(1, d), mid_layer=False,
                      d=d, out_rows=n, blk=5000)
